# Initial kernel scaffold; baseline (speedup 1.0000x reference)
import jax
import jax.numpy as jnp
from jax.experimental import pallas as pl
from jax.experimental.pallas import tpu as pltpu

_SCALE = 65536.0
_NSEG = 16
_SLOPE_MASK = jnp.int32(-8388608)
_B_MASK = jnp.int32(0x1FFFF)


def _pack_tables(intercepts, signs, exps):
    sign_bit = ((1 - signs) // 2).astype(jnp.int32)
    slope_bits = (sign_bit << 31) | ((127 + exps) << 23)
    return slope_bits | (intercepts + 65536)


def _tc_body(bp_ref, packed_ref, x_ref, o_ref):
    x = x_ref[...]
    xq = (x * _SCALE).astype(jnp.int32)
    acc = jnp.where(xq >= bp_ref[0], packed_ref[1], packed_ref[0])
    for j in range(1, _NSEG - 1):
        acc = jnp.where(xq >= bp_ref[j], packed_ref[j + 1], acc)
    slope = jax.lax.bitcast_convert_type(acc & _SLOPE_MASK, jnp.float32)
    b = (acc & _B_MASK).astype(jnp.float32) * (1.0 / _SCALE) - 1.0
    o_ref[...] = b + slope * x


def kernel(x, breakpoints, intercepts, signs, exps):
    packed = _pack_tables(intercepts, signs, exps)
    n = x.shape[0]
    cols = 1024
    rows = n // cols
    br = min(512, rows)
    grid = rows // br
    x2 = x.reshape(rows, cols)
    out = pl.pallas_call(
        _tc_body,
        grid=(grid,),
        in_specs=[
            pl.BlockSpec(memory_space=pltpu.SMEM),
            pl.BlockSpec(memory_space=pltpu.SMEM),
            pl.BlockSpec((br, cols), lambda i: (i, 0)),
        ],
        out_specs=pl.BlockSpec((br, cols), lambda i: (i, 0)),
        out_shape=jax.ShapeDtypeStruct((rows, cols), jnp.float32),
    )(breakpoints, packed, x2)
    return out.reshape(n).astype(x.dtype)


# speedup vs baseline: 3.2774x; 3.2774x over previous
import jax
import jax.numpy as jnp
from jax.experimental import pallas as pl
from jax.experimental.pallas import tpu as pltpu

_SCALE = 65536.0
_NSEG = 16
_SLOPE_MASK = -8388608
_B_MASK = 0x1FFFF


def _pack_tables(intercepts, signs, exps):
    sign_bit = ((1 - signs) // 2).astype(jnp.int32)
    slope_bits = (sign_bit << 31) | ((127 + exps) << 23)
    return slope_bits | (intercepts + 65536)


def _tc_body(bp_ref, packed_ref, x_ref, o_ref):
    x = x_ref[...]
    xq = (x * _SCALE).astype(jnp.int32)
    acc = jnp.where(xq >= bp_ref[0], packed_ref[1], packed_ref[0])
    for j in range(1, _NSEG - 1):
        acc = jnp.where(xq >= bp_ref[j], packed_ref[j + 1], acc)
    slope = jax.lax.bitcast_convert_type(acc & _SLOPE_MASK, jnp.float32)
    b = (acc & _B_MASK).astype(jnp.float32) * (1.0 / _SCALE) - 1.0
    o_ref[...] = b + slope * x


def kernel(x, breakpoints, intercepts, signs, exps):
    packed = _pack_tables(intercepts, signs, exps)
    n = x.shape[0]
    cols = 1024
    rows = n // cols
    br = min(512, rows)
    grid = rows // br
    x2 = x.reshape(rows, cols)
    out = pl.pallas_call(
        _tc_body,
        grid=(grid,),
        in_specs=[
            pl.BlockSpec(memory_space=pltpu.SMEM),
            pl.BlockSpec(memory_space=pltpu.SMEM),
            pl.BlockSpec((br, cols), lambda i: (i, 0)),
        ],
        out_specs=pl.BlockSpec((br, cols), lambda i: (i, 0)),
        out_shape=jax.ShapeDtypeStruct((rows, cols), jnp.float32),
    )(breakpoints, packed, x2)
    return out.reshape(n).astype(x.dtype)
